# baseline (speedup 1.0000x reference)
import functools

import jax
import jax.numpy as jnp
from jax import lax
from jax.experimental import pallas as pl
from jax.experimental.pallas import tpu as pltpu
from jax.experimental.pallas import tpu_sc as plsc

VOCAB = 1000000
D = 64
BATCH = 4096
SEQ = 200
NC, NS = 2, 16
NW = NC * NS
TOTAL = BATCH * SEQ
ROWS_W = TOTAL // NW
CHUNK = 128
NCHUNK = ROWS_W // CHUNK
IDX_ROWS = TOTAL // 128


def _mesh():
    return plsc.VectorSubcoreMesh(core_axis_name="c", subcore_axis_name="s")


@functools.partial(
    pl.kernel,
    mesh=_mesh(),
    out_type=jax.ShapeDtypeStruct((TOTAL, D), jnp.float32),
    scratch_types=[
        pltpu.VMEM((NCHUNK, 128), jnp.int32),
        pltpu.VMEM((CHUNK, D), jnp.float32),
        pltpu.VMEM((CHUNK, D), jnp.float32),
        pltpu.SemaphoreType.DMA,
        pltpu.SemaphoreType.DMA,
    ],
    compiler_params=pltpu.CompilerParams(use_tc_tiling_on_sc=False),
)
def _gather_kernel(table_hbm, x2_hbm, out_hbm, idx_v, buf0, buf1, sem0, sem1):
    wid = lax.axis_index("s") * NC + lax.axis_index("c")
    rbase = wid * NCHUNK
    obase = wid * ROWS_W

    pltpu.sync_copy(x2_hbm.at[pl.ds(rbase, NCHUNK)], idx_v)
    pltpu.async_copy(table_hbm.at[idx_v.at[0]], buf0, sem0)

    def body(h, carry):
        j0 = h * 2
        j1 = j0 + 1
        pltpu.async_copy(table_hbm.at[idx_v.at[j1]], buf1, sem1)
        pltpu.make_async_copy(table_hbm.at[idx_v.at[j0]], buf0, sem0).wait()
        pltpu.sync_copy(buf0, out_hbm.at[pl.ds(obase + j0 * CHUNK, CHUNK)])

        @pl.when(j1 + 1 < NCHUNK)
        def _():
            pltpu.async_copy(table_hbm.at[idx_v.at[j1 + 1]], buf0, sem0)

        pltpu.make_async_copy(table_hbm.at[idx_v.at[j1]], buf1, sem1).wait()
        pltpu.sync_copy(buf1, out_hbm.at[pl.ds(obase + j1 * CHUNK, CHUNK)])
        return carry

    lax.fori_loop(0, NCHUNK // 2, body, 0)


def kernel(x, table):
    x2 = jnp.reshape(x, (IDX_ROWS, 128))
    outf = _gather_kernel(table, x2)
    return jnp.reshape(outf, (BATCH, SEQ, D))


# speedup vs baseline: 1.0211x; 1.0211x over previous
import functools

import jax
import jax.numpy as jnp
from jax import lax
from jax.experimental import pallas as pl
from jax.experimental.pallas import tpu as pltpu
from jax.experimental.pallas import tpu_sc as plsc

VOCAB = 1000000
D = 64
BATCH = 4096
SEQ = 200
NC, NS = 2, 16
NW = NC * NS
TOTAL = BATCH * SEQ
ROWS_W = TOTAL // NW
CHUNK = 128
NCHUNK = ROWS_W // CHUNK
IDX_ROWS = TOTAL // 128


def _mesh():
    return plsc.VectorSubcoreMesh(core_axis_name="c", subcore_axis_name="s")


@functools.partial(
    pl.kernel,
    mesh=_mesh(),
    out_type=jax.ShapeDtypeStruct((TOTAL, D), jnp.float32),
    scratch_types=[
        pltpu.VMEM((NCHUNK, 128), jnp.int32),
        pltpu.VMEM((CHUNK, D), jnp.float32),
        pltpu.VMEM((CHUNK, D), jnp.float32),
        pltpu.VMEM((CHUNK, D), jnp.float32),
        pltpu.VMEM((CHUNK, D), jnp.float32),
        pltpu.SemaphoreType.DMA,
        pltpu.SemaphoreType.DMA,
        pltpu.SemaphoreType.DMA,
        pltpu.SemaphoreType.DMA,
    ],
    compiler_params=pltpu.CompilerParams(use_tc_tiling_on_sc=False),
)
def _gather_kernel(table_hbm, x2_hbm, out_hbm, idx_v,
                   buf0, buf1, buf2, buf3, sem0, sem1, sem2, sem3):
    wid = lax.axis_index("s") * NC + lax.axis_index("c")
    rbase = wid * NCHUNK
    obase = wid * ROWS_W
    bufs = [buf0, buf1, buf2, buf3]
    sems = [sem0, sem1, sem2, sem3]
    NB = 4

    pltpu.sync_copy(x2_hbm.at[pl.ds(rbase, NCHUNK)], idx_v)
    for p in range(NB - 1):
        pltpu.async_copy(table_hbm.at[idx_v.at[p]], bufs[p], sems[p])

    def body(h, carry):
        j0 = h * NB
        for b in range(NB):
            j = j0 + b
            nxt = j + (NB - 1)

            @pl.when(nxt < NCHUNK)
            def _():
                pltpu.async_copy(table_hbm.at[idx_v.at[nxt]],
                                 bufs[(b + NB - 1) % NB], sems[(b + NB - 1) % NB])

            pltpu.make_async_copy(table_hbm.at[idx_v.at[j]],
                                  bufs[b], sems[b]).wait()
            pltpu.sync_copy(bufs[b], out_hbm.at[pl.ds(obase + j * CHUNK, CHUNK)])
        return carry

    lax.fori_loop(0, NCHUNK // NB, body, 0)


def kernel(x, table):
    x2 = jnp.reshape(x, (IDX_ROWS, 128))
    outf = _gather_kernel(table, x2)
    return jnp.reshape(outf, (BATCH, SEQ, D))
